# Initial kernel scaffold; baseline (speedup 1.0000x reference)
import jax
import jax.numpy as jnp
from jax.experimental import pallas as pl


def kernel(tokens, table):
    raise NotImplementedError("write your pallas kernel here")


# speedup vs baseline: 2.9737x; 2.9737x over previous
import functools

import jax
import jax.numpy as jnp
from jax import lax
from jax.experimental import pallas as pl
from jax.experimental.pallas import tpu as pltpu
from jax.experimental.pallas import tpu_sc as plsc

_NUM_WORKERS = 32
_CHUNK = 128


def _emb_kernel(n_chunks, per_w, d,
                tok_hbm, table_hbm, out_hbm, idx_v, rows_v, gsem):
    wid = lax.axis_index("s") * 2 + lax.axis_index("c")
    row0 = wid * per_w
    pltpu.sync_copy(tok_hbm.at[wid], idx_v)

    def body(j, carry):
        pltpu.async_copy(table_hbm.at[idx_v.at[j]], rows_v.at[0], gsem).wait()
        pltpu.sync_copy(rows_v.at[0],
                        out_hbm.at[pl.ds(row0 + j * _CHUNK, _CHUNK)])
        return carry

    lax.fori_loop(0, n_chunks, body, 0)


def kernel(tokens, table):
    b, s = tokens.shape
    v, d = table.shape
    n = b * s
    assert n % (_NUM_WORKERS * _CHUNK) == 0
    per_w = n // _NUM_WORKERS
    n_chunks = per_w // _CHUNK

    tok_flat = tokens.reshape(_NUM_WORKERS, n_chunks, _CHUNK).astype(jnp.int32)

    mesh = plsc.VectorSubcoreMesh(core_axis_name="c", subcore_axis_name="s")
    run = functools.partial(
        pl.kernel,
        mesh=mesh,
        out_type=jax.ShapeDtypeStruct((n, d), jnp.float32),
        scratch_types=[
            pltpu.VMEM((n_chunks, _CHUNK), jnp.int32),
            pltpu.VMEM((1, _CHUNK, d), jnp.float32),
            pltpu.SemaphoreType.DMA,
        ],
    )(functools.partial(_emb_kernel, n_chunks, per_w, d))
    out = run(tok_flat, table)
    return out.reshape(b, s, d)


# speedup vs baseline: 3.1336x; 1.0538x over previous
import functools

import jax
import jax.numpy as jnp
from jax import lax
from jax.experimental import pallas as pl
from jax.experimental.pallas import tpu as pltpu
from jax.experimental.pallas import tpu_sc as plsc

_NUM_WORKERS = 32
_CHUNK = 128


def _emb_kernel(n_chunks, per_w, d,
                tok_hbm, table_hbm, out_hbm, idx_v, rows_v, gsem, wsem):
    wid = lax.axis_index("s") * 2 + lax.axis_index("c")
    row0 = wid * per_w
    pltpu.sync_copy(tok_hbm.at[wid], idx_v)

    def gather(j, p):
        return pltpu.make_async_copy(
            table_hbm.at[idx_v.at[j]], rows_v.at[p], gsem)

    def wback(j, p):
        return pltpu.make_async_copy(
            rows_v.at[p], out_hbm.at[pl.ds(row0 + j * _CHUNK, _CHUNK)], wsem)

    gather(0, 0).start()
    gather(0, 0).wait()
    wback(0, 0).start()
    gather(1, 1).start()

    def body(j, carry):
        p = lax.rem(j, 2)
        gather(j, p).wait()
        wback(j, p).start()
        wback(j - 1, 1 - p).wait()
        gather(j + 1, 1 - p).start()
        return carry

    lax.fori_loop(1, n_chunks - 1, body, 0)

    last = n_chunks - 1
    lp = last % 2
    gather(last, lp).wait()
    wback(last, lp).start()
    wback(last - 1, 1 - lp).wait()
    wback(last, lp).wait()


def kernel(tokens, table):
    b, s = tokens.shape
    v, d = table.shape
    n = b * s
    assert n % (_NUM_WORKERS * _CHUNK) == 0
    per_w = n // _NUM_WORKERS
    n_chunks = per_w // _CHUNK

    tok_flat = tokens.reshape(_NUM_WORKERS, n_chunks, _CHUNK).astype(jnp.int32)

    mesh = plsc.VectorSubcoreMesh(core_axis_name="c", subcore_axis_name="s")
    run = functools.partial(
        pl.kernel,
        mesh=mesh,
        out_type=jax.ShapeDtypeStruct((n, d), jnp.float32),
        scratch_types=[
            pltpu.VMEM((n_chunks, _CHUNK), jnp.int32),
            pltpu.VMEM((2, _CHUNK, d), jnp.float32),
            pltpu.SemaphoreType.DMA,
            pltpu.SemaphoreType.DMA,
        ],
    )(functools.partial(_emb_kernel, n_chunks, per_w, d))
    out = run(tok_flat, table)
    return out.reshape(b, s, d)


# speedup vs baseline: 3.3591x; 1.0720x over previous
import functools

import jax
import jax.numpy as jnp
from jax import lax
from jax.experimental import pallas as pl
from jax.experimental.pallas import tpu as pltpu
from jax.experimental.pallas import tpu_sc as plsc

_NUM_WORKERS = 32
_CHUNK = 128


def _emb_kernel(n_chunks, per_w, d,
                tok_hbm, table_hbm, out_hbm, idx_v, rows_v, gsem, wsem):
    wid = lax.axis_index("s") * 2 + lax.axis_index("c")
    row0 = wid * per_w
    pltpu.sync_copy(tok_hbm.at[wid], idx_v)

    def gather(j, p):
        return pltpu.make_async_copy(
            table_hbm.at[idx_v.at[j]], rows_v.at[p], gsem)

    def wback(j, p):
        return pltpu.make_async_copy(
            rows_v.at[p], out_hbm.at[pl.ds(row0 + j * _CHUNK, _CHUNK)], wsem)

    nbuf = 4
    depth = nbuf - 1

    for k in range(depth):
        gather(k, k).start()

    gather(0, 0).wait()
    wback(0, 0).start()
    gather(depth, depth).start()

    def body(j, carry):
        p = lax.rem(j, nbuf)
        pm1 = lax.rem(j + (nbuf - 1), nbuf)
        gather(j, p).wait()
        wback(j, p).start()
        wback(j - 1, pm1).wait()
        gather(j + depth, pm1).start()
        return carry

    lax.fori_loop(1, n_chunks - depth, body, 0)

    for j in range(n_chunks - depth, n_chunks):
        gather(j, j % nbuf).wait()
        wback(j, j % nbuf).start()
        wback(j - 1, (j - 1) % nbuf).wait()
    wback(n_chunks - 1, (n_chunks - 1) % nbuf).wait()


def kernel(tokens, table):
    b, s = tokens.shape
    v, d = table.shape
    n = b * s
    assert n % (_NUM_WORKERS * _CHUNK) == 0
    per_w = n // _NUM_WORKERS
    n_chunks = per_w // _CHUNK

    tok_flat = tokens.reshape(_NUM_WORKERS, n_chunks, _CHUNK).astype(jnp.int32)

    mesh = plsc.VectorSubcoreMesh(core_axis_name="c", subcore_axis_name="s")
    run = functools.partial(
        pl.kernel,
        mesh=mesh,
        out_type=jax.ShapeDtypeStruct((n, d), jnp.float32),
        scratch_types=[
            pltpu.VMEM((n_chunks, _CHUNK), jnp.int32),
            pltpu.VMEM((4, _CHUNK, d), jnp.float32),
            pltpu.SemaphoreType.DMA,
            pltpu.SemaphoreType.DMA,
        ],
    )(functools.partial(_emb_kernel, n_chunks, per_w, d))
    out = run(tok_flat, table)
    return out.reshape(b, s, d)


# speedup vs baseline: 5.9929x; 1.7841x over previous
import functools

import jax
import jax.numpy as jnp
from jax import lax
from jax.experimental import pallas as pl
from jax.experimental.pallas import tpu as pltpu
from jax.experimental.pallas import tpu_sc as plsc

_NUM_WORKERS = 32
_NBUF = 6


def _emb_kernel(per_w, s, d,
                tok_hbm, table_hbm, out_hbm, idx_v, rows_v, gsem, wsem):
    wid = lax.axis_index("s") * 2 + lax.axis_index("c")
    b0 = wid * per_w
    pltpu.sync_copy(tok_hbm.at[wid], idx_v)

    def gather(j, p):
        return pltpu.make_async_copy(
            table_hbm.at[idx_v.at[j]], rows_v.at[p], gsem)

    def wback(j, p):
        return pltpu.make_async_copy(rows_v.at[p], out_hbm.at[b0 + j], wsem)

    depth = _NBUF - 1

    for k in range(depth):
        gather(k, k).start()

    gather(0, 0).wait()
    wback(0, 0).start()
    gather(depth, depth).start()

    def body(j, carry):
        p = lax.rem(j, _NBUF)
        pm1 = lax.rem(j + (_NBUF - 1), _NBUF)
        gather(j, p).wait()
        wback(j, p).start()
        wback(j - 1, pm1).wait()
        gather(j + depth, pm1).start()
        return carry

    lax.fori_loop(1, per_w - depth, body, 0)

    for j in range(per_w - depth, per_w):
        gather(j, j % _NBUF).wait()
        wback(j, j % _NBUF).start()
        wback(j - 1, (j - 1) % _NBUF).wait()
    wback(per_w - 1, (per_w - 1) % _NBUF).wait()


def kernel(tokens, table):
    b, s = tokens.shape
    v, d = table.shape
    assert b % _NUM_WORKERS == 0
    per_w = b // _NUM_WORKERS

    tok_3d = tokens.reshape(_NUM_WORKERS, per_w, s).astype(jnp.int32)

    mesh = plsc.VectorSubcoreMesh(core_axis_name="c", subcore_axis_name="s")
    run = functools.partial(
        pl.kernel,
        mesh=mesh,
        out_type=jax.ShapeDtypeStruct((b, s, d), jnp.float32),
        scratch_types=[
            pltpu.VMEM((per_w, s), jnp.int32),
            pltpu.VMEM((_NBUF, s, d), jnp.float32),
            pltpu.SemaphoreType.DMA,
            pltpu.SemaphoreType.DMA,
        ],
    )(functools.partial(_emb_kernel, per_w, s, d))
    return run(tok_3d, table)


# speedup vs baseline: 10.8211x; 1.8056x over previous
import functools

import jax
import jax.numpy as jnp
from jax import lax
from jax.experimental import pallas as pl
from jax.experimental.pallas import tpu as pltpu
from jax.experimental.pallas import tpu_sc as plsc

_NUM_WORKERS = 32
_NBUF = 6


def _emb_kernel(n_chunks, per_w,
                tok_hbm, table_hbm, out_hbm, idx_v, rows_v, gsem, wsem):
    wid = lax.axis_index("s") * 2 + lax.axis_index("c")
    b0 = wid * per_w
    pltpu.sync_copy(tok_hbm.at[:, pl.ds(b0, per_w)], idx_v)

    def gather(j, p):
        return pltpu.make_async_copy(
            table_hbm.at[idx_v.at[j]], rows_v.at[p], gsem)

    def wback(j, p):
        return pltpu.make_async_copy(
            rows_v.at[p], out_hbm.at[j, pl.ds(b0, per_w)], wsem)

    depth = _NBUF - 1

    for k in range(depth):
        gather(k, k).start()

    gather(0, 0).wait()
    wback(0, 0).start()
    gather(depth, depth).start()

    def body(j, carry):
        p = lax.rem(j, _NBUF)
        pm1 = lax.rem(j + (_NBUF - 1), _NBUF)
        gather(j, p).wait()
        wback(j, p).start()
        wback(j - 1, pm1).wait()
        gather(j + depth, pm1).start()
        return carry

    lax.fori_loop(1, n_chunks - depth, body, 0)

    for j in range(n_chunks - depth, n_chunks):
        gather(j, j % _NBUF).wait()
        wback(j, j % _NBUF).start()
        wback(j - 1, (j - 1) % _NBUF).wait()
    wback(n_chunks - 1, (n_chunks - 1) % _NBUF).wait()


def kernel(tokens, table):
    b, s = tokens.shape
    v, d = table.shape
    assert b % _NUM_WORKERS == 0
    per_w = b // _NUM_WORKERS

    tok_t = tokens.astype(jnp.int32).T

    mesh = plsc.VectorSubcoreMesh(core_axis_name="c", subcore_axis_name="s")
    run = functools.partial(
        pl.kernel,
        mesh=mesh,
        out_type=jax.ShapeDtypeStruct((s, b, d), jnp.float32),
        scratch_types=[
            pltpu.VMEM((s, per_w), jnp.int32),
            pltpu.VMEM((_NBUF, per_w, d), jnp.float32),
            pltpu.SemaphoreType.DMA,
            pltpu.SemaphoreType.DMA,
        ],
    )(functools.partial(_emb_kernel, s, per_w))
    out3 = run(tok_t, table)
    return jnp.transpose(out3, (1, 0, 2))
